# Initial kernel scaffold; baseline (speedup 1.0000x reference)
import functools

import jax
import jax.numpy as jnp
from jax import lax
from jax.experimental import pallas as pl
from jax.experimental.pallas import tpu as pltpu
from jax.experimental.pallas import tpu_sc as plsc

N = 10000
E = 320000
D = 128
C = 7
L = 16
NC, NS = 2, 16
NW = NC * NS
CH = 128
CPT = 79
E_PAD = NW * CPT * CH
N_PAD = 10240
RPT = N_PAD // NS
BN = 512


def _make_sc_pass(gather: bool):
  mesh = plsc.VectorSubcoreMesh(core_axis_name="c", subcore_axis_name="s")

  @functools.partial(
      pl.kernel,
      out_type=jax.ShapeDtypeStruct((NC, N_PAD, L), jnp.float32),
      mesh=mesh,
      scratch_types=[
          pltpu.VMEM((CPT, CH), jnp.int32),
          pltpu.VMEM((CPT, CH), jnp.int32),
          pltpu.VMEM((CH, L), jnp.float32),
          pltpu.VMEM_SHARED((N_PAD, L), jnp.float32),
          pltpu.SemaphoreType.DMA,
      ],
  )
  def sc_pass(u_hbm, src_hbm, dst_hbm, out_hbm, src_v, dst_v, rows_v, acc,
              sem):
    cid = lax.axis_index("c")
    sid = lax.axis_index("s")
    wid = cid * NS + sid

    @pl.loop(0, CH)
    def _(i):
      rows_v[i, :] = jnp.zeros((L,), jnp.float32)

    @pl.loop(0, RPT // CH)
    def _(k):
      pltpu.sync_copy(rows_v, acc.at[pl.ds(sid * RPT + k * CH, CH)])

    if not gather:
      @pl.loop(0, CH)
      def _(i):
        rows_v[i, :] = jnp.ones((L,), jnp.float32)

    pltpu.sync_copy(src_hbm.at[wid], src_v)
    pltpu.sync_copy(dst_hbm.at[wid], dst_v)
    plsc.subcore_barrier()

    @pl.loop(0, CPT)
    def _(j):
      if gather:
        pltpu.async_copy(u_hbm.at[src_v.at[j]], rows_v, sem).wait()
      pltpu.sync_copy(rows_v, acc.at[dst_v.at[j]], add=True)

    plsc.subcore_barrier()
    pltpu.sync_copy(acc.at[pl.ds(sid * RPT, RPT)],
                    out_hbm.at[cid, pl.ds(sid * RPT, RPT)])

  return sc_pass


_sc_hop = _make_sc_pass(gather=True)
_sc_deg = _make_sc_pass(gather=False)


def _row_specs(n):
  return [pl.BlockSpec((BN, L), lambda i: (i, 0)) for _ in range(n)]


def _tc_pre(x_pad, wp, d0, d1):
  def body(x_ref, w_ref, d0_ref, d1_ref, u0_ref, inv_ref, dis_ref):
    deg = d0_ref[...] + d1_ref[...] + 1.0
    inv = 1.0 / deg
    dis = lax.rsqrt(deg)
    z = jnp.dot(x_ref[...], w_ref[...], preferred_element_type=jnp.float32)
    u0_ref[...] = dis * z
    inv_ref[...] = inv
    dis_ref[...] = dis

  return pl.pallas_call(
      body,
      grid=(N_PAD // BN,),
      in_specs=[pl.BlockSpec((BN, D), lambda i: (i, 0)),
                pl.BlockSpec((D, L), lambda i: (0, 0))] + _row_specs(2),
      out_specs=_row_specs(3),
      out_shape=[jax.ShapeDtypeStruct((N_PAD, L), jnp.float32)] * 3,
  )(x_pad, wp, d0, d1)


def _tc_mid(a0, a1, u0, inv):
  def body(a0_ref, a1_ref, u0_ref, inv_ref, u1_ref):
    u1_ref[...] = inv_ref[...] * (a0_ref[...] + a1_ref[...] + u0_ref[...])

  return pl.pallas_call(
      body,
      grid=(N_PAD // BN,),
      in_specs=_row_specs(4),
      out_specs=_row_specs(1)[0],
      out_shape=jax.ShapeDtypeStruct((N_PAD, L), jnp.float32),
  )(a0, a1, u0, inv)


def _tc_post(a0, a1, u1, dis, b16):
  def body(a0_ref, a1_ref, u1_ref, dis_ref, b_ref, o_ref):
    h2 = dis_ref[...] * (a0_ref[...] + a1_ref[...] + u1_ref[...])
    logits = h2 + b_ref[...]
    col = lax.broadcasted_iota(jnp.int32, (BN, L), 1)
    valid = col < C
    masked = jnp.where(valid, logits, jnp.float32(-1e30))
    m = jnp.max(masked, axis=1, keepdims=True)
    s = jnp.sum(jnp.where(valid, jnp.exp(logits - m), 0.0), axis=1,
                keepdims=True)
    o_ref[...] = logits - m - jnp.log(s)

  return pl.pallas_call(
      body,
      grid=(N_PAD // BN,),
      in_specs=_row_specs(3) + [pl.BlockSpec((BN, L), lambda i: (i, 0)),
                                pl.BlockSpec((1, L), lambda i: (0, 0))],
      out_specs=_row_specs(1)[0],
      out_shape=jax.ShapeDtypeStruct((N_PAD, L), jnp.float32),
  )(a0, a1, u1, dis, b16)


def kernel(x, edge_index, W, b):
  x_pad = jnp.pad(x, ((0, N_PAD - N), (0, 0)))
  wp = jnp.pad(W.T.astype(jnp.float32), ((0, 0), (0, L - C)))
  b16 = jnp.pad(b.astype(jnp.float32), (0, L - C)).reshape(1, L)
  src = jnp.pad(edge_index[0], (0, E_PAD - E), constant_values=N)
  dst = jnp.pad(edge_index[1], (0, E_PAD - E), constant_values=N)
  src3 = src.reshape(NW, CPT, CH)
  dst3 = dst.reshape(NW, CPT, CH)

  dummy_u = jnp.zeros((N_PAD, L), jnp.float32)
  degp = _sc_deg(dummy_u, src3, dst3)
  u0, inv, dis = _tc_pre(x_pad, wp, degp[0], degp[1])
  a1 = _sc_hop(u0, src3, dst3)
  u1 = _tc_mid(a1[0], a1[1], u0, inv)
  a2 = _sc_hop(u1, src3, dst3)
  out = _tc_post(a2[0], a2[1], u1, dis, b16)
  return out[:N, :C]


# speedup vs baseline: 29.7187x; 29.7187x over previous
import functools

import jax
import jax.numpy as jnp
from jax import lax
from jax.experimental import pallas as pl
from jax.experimental.pallas import tpu as pltpu
from jax.experimental.pallas import tpu_sc as plsc

N = 10000
E = 320000
D = 128
C = 7
L = 16
NC, NS = 2, 16
NW = NC * NS
CH = 128
CPT = 79
E_PAD = NW * CPT * CH
N_PAD = 10240
RPT = N_PAD // NS
BN = 512


def _make_sc_pass(gather: bool):
  mesh = plsc.VectorSubcoreMesh(core_axis_name="c", subcore_axis_name="s",
                                num_cores=NC, num_subcores=NS)

  @functools.partial(
      pl.kernel,
      out_type=jax.ShapeDtypeStruct((NC, N_PAD, L), jnp.float32),
      mesh=mesh,
      compiler_params=pltpu.CompilerParams(use_tc_tiling_on_sc=False),
      scratch_types=[
          pltpu.VMEM((CPT, CH), jnp.int32),
          pltpu.VMEM((CPT, CH), jnp.int32),
          pltpu.VMEM((CH, L), jnp.float32),
          pltpu.VMEM_SHARED((N_PAD, L), jnp.float32),
          pltpu.SemaphoreType.DMA,
      ],
  )
  def sc_pass(u_hbm, src_hbm, dst_hbm, out_hbm, src_v, dst_v, rows_v, acc,
              sem):
    cid = lax.axis_index("c")
    sid = lax.axis_index("s")
    wid = cid * NS + sid

    @pl.loop(0, CH)
    def _(i):
      rows_v[i, :] = jnp.zeros((L,), jnp.float32)

    @pl.loop(0, RPT // CH)
    def _(k):
      pltpu.sync_copy(rows_v, acc.at[pl.ds(sid * RPT + k * CH, CH)])

    if not gather:
      @pl.loop(0, CH)
      def _(i):
        rows_v[i, :] = jnp.ones((L,), jnp.float32)

    pltpu.sync_copy(src_hbm.at[wid], src_v)
    pltpu.sync_copy(dst_hbm.at[wid], dst_v)
    plsc.subcore_barrier()

    @pl.loop(0, CPT)
    def _(j):
      if gather:
        pltpu.async_copy(u_hbm.at[src_v.at[j]], rows_v, sem).wait()
      pltpu.sync_copy(rows_v, acc.at[dst_v.at[j]], add=True)

    plsc.subcore_barrier()
    pltpu.sync_copy(acc.at[pl.ds(sid * RPT, RPT)],
                    out_hbm.at[cid, pl.ds(sid * RPT, RPT)])

  return sc_pass


_sc_hop = _make_sc_pass(gather=True)
_sc_deg = _make_sc_pass(gather=False)


def _row_specs(n):
  return [pl.BlockSpec((BN, L), lambda i: (i, 0)) for _ in range(n)]


def _tc_pre(x_pad, wp, d0, d1):
  def body(x_ref, w_ref, d0_ref, d1_ref, u0_ref, inv_ref, dis_ref):
    deg = d0_ref[...] + d1_ref[...] + 1.0
    inv = 1.0 / deg
    dis = lax.rsqrt(deg)
    z = jnp.dot(x_ref[...], w_ref[...], preferred_element_type=jnp.float32)
    u0_ref[...] = dis * z
    inv_ref[...] = inv
    dis_ref[...] = dis

  return pl.pallas_call(
      body,
      grid=(N_PAD // BN,),
      in_specs=[pl.BlockSpec((BN, D), lambda i: (i, 0)),
                pl.BlockSpec((D, L), lambda i: (0, 0))] + _row_specs(2),
      out_specs=_row_specs(3),
      out_shape=[jax.ShapeDtypeStruct((N_PAD, L), jnp.float32)] * 3,
  )(x_pad, wp, d0, d1)


def _tc_mid(a0, a1, u0, inv):
  def body(a0_ref, a1_ref, u0_ref, inv_ref, u1_ref):
    u1_ref[...] = inv_ref[...] * (a0_ref[...] + a1_ref[...] + u0_ref[...])

  return pl.pallas_call(
      body,
      grid=(N_PAD // BN,),
      in_specs=_row_specs(4),
      out_specs=_row_specs(1)[0],
      out_shape=jax.ShapeDtypeStruct((N_PAD, L), jnp.float32),
  )(a0, a1, u0, inv)


def _tc_post(a0, a1, u1, dis, b16):
  def body(a0_ref, a1_ref, u1_ref, dis_ref, b_ref, o_ref):
    h2 = dis_ref[...] * (a0_ref[...] + a1_ref[...] + u1_ref[...])
    logits = h2 + b_ref[...]
    col = lax.broadcasted_iota(jnp.int32, (BN, L), 1)
    valid = col < C
    masked = jnp.where(valid, logits, jnp.float32(-1e30))
    m = jnp.max(masked, axis=1, keepdims=True)
    s = jnp.sum(jnp.where(valid, jnp.exp(logits - m), 0.0), axis=1,
                keepdims=True)
    o_ref[...] = logits - m - jnp.log(s)

  return pl.pallas_call(
      body,
      grid=(N_PAD // BN,),
      in_specs=_row_specs(3) + [pl.BlockSpec((BN, L), lambda i: (i, 0)),
                                pl.BlockSpec((1, L), lambda i: (0, 0))],
      out_specs=_row_specs(1)[0],
      out_shape=jax.ShapeDtypeStruct((N_PAD, L), jnp.float32),
  )(a0, a1, u1, dis, b16)


def kernel(x, edge_index, W, b):
  x_pad = jnp.pad(x, ((0, N_PAD - N), (0, 0)))
  wp = jnp.pad(W.T.astype(jnp.float32), ((0, 0), (0, L - C)))
  b16 = jnp.pad(b.astype(jnp.float32), (0, L - C)).reshape(1, L)
  src = jnp.pad(edge_index[0], (0, E_PAD - E), constant_values=N)
  dst = jnp.pad(edge_index[1], (0, E_PAD - E), constant_values=N)
  src3 = src.reshape(NW, CPT, CH)
  dst3 = dst.reshape(NW, CPT, CH)

  dummy_u = jnp.zeros((N_PAD, L), jnp.float32)
  degp = _sc_deg(dummy_u, src3, dst3)
  u0, inv, dis = _tc_pre(x_pad, wp, degp[0], degp[1])
  a1 = _sc_hop(u0, src3, dst3)
  u1 = _tc_mid(a1[0], a1[1], u0, inv)
  a2 = _sc_hop(u1, src3, dst3)
  out = _tc_post(a2[0], a2[1], u1, dis, b16)
  return out[:N, :C]


# speedup vs baseline: 32.2268x; 1.0844x over previous
import functools

import jax
import jax.numpy as jnp
from jax import lax
from jax.experimental import pallas as pl
from jax.experimental.pallas import tpu as pltpu
from jax.experimental.pallas import tpu_sc as plsc

N = 10000
E = 320000
D = 128
C = 7
L = 16
NC, NS = 2, 16
NW = NC * NS
CH = 128
CPT = 80
KG = 16
NG = CPT // KG
E_PAD = NW * CPT * CH
N_PAD = 10240
RPT = N_PAD // NS
BN = 512


def _make_sc_pass(gather: bool):
  mesh = plsc.VectorSubcoreMesh(core_axis_name="c", subcore_axis_name="s",
                                num_cores=NC, num_subcores=NS)

  @functools.partial(
      pl.kernel,
      out_type=jax.ShapeDtypeStruct((NC, N_PAD, L), jnp.float32),
      mesh=mesh,
      compiler_params=pltpu.CompilerParams(use_tc_tiling_on_sc=False),
      scratch_types=[
          pltpu.VMEM((CPT, CH), jnp.int32),
          pltpu.VMEM((CPT, CH), jnp.int32),
          pltpu.VMEM((KG, CH, L), jnp.float32),
          pltpu.VMEM_SHARED((N_PAD, L), jnp.float32),
          pltpu.SemaphoreType.DMA,
          pltpu.SemaphoreType.DMA,
      ],
  )
  def sc_pass(u_hbm, src_hbm, dst_hbm, out_hbm, src_v, dst_v, rows_v, acc,
              sem_g, sem_s):
    cid = lax.axis_index("c")
    sid = lax.axis_index("s")
    wid = cid * NS + sid

    ds_src = pltpu.async_copy(src_hbm.at[wid], src_v, sem_g)
    ds_dst = pltpu.async_copy(dst_hbm.at[wid], dst_v, sem_s)

    @pl.loop(0, CH)
    def _(i):
      rows_v[0, i, :] = jnp.zeros((L,), jnp.float32)

    @pl.loop(0, RPT // CH)
    def _(k):
      pltpu.sync_copy(rows_v.at[0], acc.at[pl.ds(sid * RPT + k * CH, CH)])

    if not gather:
      @pl.loop(0, CH)
      def _(i):
        rows_v[0, i, :] = jnp.ones((L,), jnp.float32)

    ds_src.wait()
    ds_dst.wait()
    plsc.subcore_barrier()

    @pl.loop(0, NG)
    def _(g):
      base = g * KG
      if gather:
        gds = [pltpu.async_copy(u_hbm.at[src_v.at[base + k]], rows_v.at[k],
                                sem_g) for k in range(KG)]
        for d in gds:
          d.wait()
        sds = [pltpu.async_copy(rows_v.at[k], acc.at[dst_v.at[base + k]],
                                sem_s, add=True) for k in range(KG)]
      else:
        sds = [pltpu.async_copy(rows_v.at[0], acc.at[dst_v.at[base + k]],
                                sem_s, add=True) for k in range(KG)]
      for d in sds:
        d.wait()

    plsc.subcore_barrier()
    pltpu.sync_copy(acc.at[pl.ds(sid * RPT, RPT)],
                    out_hbm.at[cid, pl.ds(sid * RPT, RPT)])

  return sc_pass


_sc_hop = _make_sc_pass(gather=True)
_sc_deg = _make_sc_pass(gather=False)


def _row_specs(n):
  return [pl.BlockSpec((BN, L), lambda i: (i, 0)) for _ in range(n)]


def _tc_pre(x_pad, wp, d0, d1):
  def body(x_ref, w_ref, d0_ref, d1_ref, u0_ref, inv_ref, dis_ref):
    deg = d0_ref[...] + d1_ref[...] + 1.0
    inv = 1.0 / deg
    dis = lax.rsqrt(deg)
    z = jnp.dot(x_ref[...], w_ref[...], preferred_element_type=jnp.float32)
    u0_ref[...] = dis * z
    inv_ref[...] = inv
    dis_ref[...] = dis

  return pl.pallas_call(
      body,
      grid=(N_PAD // BN,),
      in_specs=[pl.BlockSpec((BN, D), lambda i: (i, 0)),
                pl.BlockSpec((D, L), lambda i: (0, 0))] + _row_specs(2),
      out_specs=_row_specs(3),
      out_shape=[jax.ShapeDtypeStruct((N_PAD, L), jnp.float32)] * 3,
  )(x_pad, wp, d0, d1)


def _tc_mid(a0, a1, u0, inv):
  def body(a0_ref, a1_ref, u0_ref, inv_ref, u1_ref):
    u1_ref[...] = inv_ref[...] * (a0_ref[...] + a1_ref[...] + u0_ref[...])

  return pl.pallas_call(
      body,
      grid=(N_PAD // BN,),
      in_specs=_row_specs(4),
      out_specs=_row_specs(1)[0],
      out_shape=jax.ShapeDtypeStruct((N_PAD, L), jnp.float32),
  )(a0, a1, u0, inv)


def _tc_post(a0, a1, u1, dis, b16):
  def body(a0_ref, a1_ref, u1_ref, dis_ref, b_ref, o_ref):
    h2 = dis_ref[...] * (a0_ref[...] + a1_ref[...] + u1_ref[...])
    logits = h2 + b_ref[...]
    col = lax.broadcasted_iota(jnp.int32, (BN, L), 1)
    valid = col < C
    masked = jnp.where(valid, logits, jnp.float32(-1e30))
    m = jnp.max(masked, axis=1, keepdims=True)
    s = jnp.sum(jnp.where(valid, jnp.exp(logits - m), 0.0), axis=1,
                keepdims=True)
    o_ref[...] = logits - m - jnp.log(s)

  return pl.pallas_call(
      body,
      grid=(N_PAD // BN,),
      in_specs=_row_specs(3) + [pl.BlockSpec((BN, L), lambda i: (i, 0)),
                                pl.BlockSpec((1, L), lambda i: (0, 0))],
      out_specs=_row_specs(1)[0],
      out_shape=jax.ShapeDtypeStruct((N_PAD, L), jnp.float32),
  )(a0, a1, u1, dis, b16)


def kernel(x, edge_index, W, b):
  x_pad = jnp.pad(x, ((0, N_PAD - N), (0, 0)))
  wp = jnp.pad(W.T.astype(jnp.float32), ((0, 0), (0, L - C)))
  b16 = jnp.pad(b.astype(jnp.float32), (0, L - C)).reshape(1, L)
  src = jnp.pad(edge_index[0], (0, E_PAD - E), constant_values=N)
  dst = jnp.pad(edge_index[1], (0, E_PAD - E), constant_values=N)
  src3 = src.reshape(NW, CPT, CH)
  dst3 = dst.reshape(NW, CPT, CH)

  dummy_u = jnp.zeros((N_PAD, L), jnp.float32)
  degp = _sc_deg(dummy_u, src3, dst3)
  u0, inv, dis = _tc_pre(x_pad, wp, degp[0], degp[1])
  a1 = _sc_hop(u0, src3, dst3)
  u1 = _tc_mid(a1[0], a1[1], u0, inv)
  a2 = _sc_hop(u1, src3, dst3)
  out = _tc_post(a2[0], a2[1], u1, dis, b16)
  return out[:N, :C]


# speedup vs baseline: 44.8679x; 1.3923x over previous
import functools

import jax
import jax.numpy as jnp
from jax import lax
from jax.experimental import pallas as pl
from jax.experimental.pallas import tpu as pltpu
from jax.experimental.pallas import tpu_sc as plsc

N = 10000
E = 320000
D = 128
C = 7
L = 16
NC, NS = 2, 16
NW = NC * NS
CH = 128
CPT = 80
KG = 16
NG = CPT // KG
E_PAD = NW * CPT * CH
N_PAD = 10240
RPT = N_PAD // NS
BN = 512


def _make_sc_pass(gather: bool):
  mesh = plsc.VectorSubcoreMesh(core_axis_name="c", subcore_axis_name="s",
                                num_cores=NC, num_subcores=NS)

  @functools.partial(
      pl.kernel,
      out_type=jax.ShapeDtypeStruct((NC, N_PAD, L), jnp.float32),
      mesh=mesh,
      compiler_params=pltpu.CompilerParams(use_tc_tiling_on_sc=False),
      scratch_types=[
          pltpu.VMEM((CPT, CH), jnp.int32),
          pltpu.VMEM((CPT, CH), jnp.int32),
          pltpu.VMEM((KG, CH, L), jnp.float32),
          pltpu.VMEM_SHARED((N_PAD, L), jnp.float32),
          pltpu.SemaphoreType.DMA,
          pltpu.SemaphoreType.DMA,
      ],
  )
  def sc_pass(u_hbm, src_hbm, dst_hbm, out_hbm, src_v, dst_v, rows_v, acc,
              sem_g, sem_s):
    cid = lax.axis_index("c")
    sid = lax.axis_index("s")
    wid = cid * NS + sid

    ds_src = pltpu.async_copy(src_hbm.at[wid], src_v, sem_g)
    ds_dst = pltpu.async_copy(dst_hbm.at[wid], dst_v, sem_s)

    @pl.loop(0, CH)
    def _(i):
      rows_v[0, i, :] = jnp.zeros((L,), jnp.float32)

    @pl.loop(0, RPT // CH)
    def _(k):
      pltpu.sync_copy(rows_v.at[0], acc.at[pl.ds(sid * RPT + k * CH, CH)])

    if not gather:
      @pl.loop(0, CH)
      def _(i):
        rows_v[0, i, :] = jnp.ones((L,), jnp.float32)

    ds_src.wait()
    ds_dst.wait()
    plsc.subcore_barrier()

    @pl.loop(0, NG)
    def _(g):
      base = g * KG
      if gather:
        gds = [pltpu.async_copy(u_hbm.at[src_v.at[base + k]], rows_v.at[k],
                                sem_g) for k in range(KG)]
        for d in gds:
          d.wait()
        sds = [pltpu.async_copy(rows_v.at[k], acc.at[dst_v.at[base + k]],
                                sem_s, add=True) for k in range(KG)]
      else:
        sds = [pltpu.async_copy(rows_v.at[0], acc.at[dst_v.at[base + k]],
                                sem_s, add=True) for k in range(KG)]
      for d in sds:
        d.wait()

    plsc.subcore_barrier()
    pltpu.sync_copy(acc.at[pl.ds(sid * RPT, RPT)],
                    out_hbm.at[cid, pl.ds(sid * RPT, RPT)])

  return sc_pass


_sc_deg = _make_sc_pass(gather=False)


def _make_sc_hop(combine: bool):
  mesh = plsc.VectorSubcoreMesh(core_axis_name="c", subcore_axis_name="s",
                                num_cores=NC, num_subcores=NS)
  agg_t = jax.ShapeDtypeStruct((NC, N_PAD, L), jnp.float32)
  out_type = [agg_t, jax.ShapeDtypeStruct((N_PAD, L), jnp.float32)] \
      if combine else agg_t
  nwork = 4 if combine else 0

  @functools.partial(
      pl.kernel,
      out_type=out_type,
      mesh=mesh,
      compiler_params=pltpu.CompilerParams(use_tc_tiling_on_sc=False),
      scratch_types=[
          pltpu.VMEM((CPT, CH), jnp.int32),
          pltpu.VMEM((CPT, CH), jnp.int32),
          pltpu.VMEM((KG, CH, L), jnp.float32),
          pltpu.VMEM((max(nwork, 1), RPT, L), jnp.float32),
          pltpu.VMEM_SHARED((N_PAD, L), jnp.float32),
          pltpu.VMEM_SHARED((N_PAD, L), jnp.float32),
          pltpu.SemaphoreType.DMA,
          pltpu.SemaphoreType.DMA,
      ],
  )
  def sc_hop(*refs):
    if combine:
      (a0_hbm, a1_hbm, u_hbm, inv_hbm, src_hbm, dst_hbm, agg_out, u_out,
       src_v, dst_v, rows_v, work_v, u_spm, acc, sem_g, sem_s) = refs
    else:
      (u_hbm, src_hbm, dst_hbm, agg_out,
       src_v, dst_v, rows_v, work_v, u_spm, acc, sem_g, sem_s) = refs
    cid = lax.axis_index("c")
    sid = lax.axis_index("s")
    wid = cid * NS + sid
    slab = pl.ds(sid * RPT, RPT)

    ds_src = pltpu.async_copy(src_hbm.at[wid], src_v, sem_g)
    ds_dst = pltpu.async_copy(dst_hbm.at[wid], dst_v, sem_s)

    if combine:
      pltpu.sync_copy(a0_hbm.at[slab], work_v.at[0])
      pltpu.sync_copy(a1_hbm.at[slab], work_v.at[1])
      pltpu.sync_copy(u_hbm.at[slab], work_v.at[2])
      pltpu.sync_copy(inv_hbm.at[slab], work_v.at[3])

      @pl.loop(0, RPT)
      def _(i):
        work_v[0, i, :] = work_v[3, i, :] * (
            work_v[0, i, :] + work_v[1, i, :] + work_v[2, i, :])

      pltpu.sync_copy(work_v.at[0], u_spm.at[slab])

      @pl.when(cid == 0)
      def _():
        pltpu.sync_copy(work_v.at[0], u_out.at[slab])
    else:
      pltpu.sync_copy(u_hbm.at[slab], u_spm.at[slab])

    @pl.loop(0, CH)
    def _(i):
      rows_v[0, i, :] = jnp.zeros((L,), jnp.float32)

    @pl.loop(0, RPT // CH)
    def _(k):
      pltpu.sync_copy(rows_v.at[0], acc.at[pl.ds(sid * RPT + k * CH, CH)])

    ds_src.wait()
    ds_dst.wait()
    plsc.subcore_barrier()

    @pl.loop(0, NG)
    def _(g):
      base = g * KG
      gds = [pltpu.async_copy(u_spm.at[src_v.at[base + k]], rows_v.at[k],
                              sem_g) for k in range(KG)]
      for d in gds:
        d.wait()
      sds = [pltpu.async_copy(rows_v.at[k], acc.at[dst_v.at[base + k]],
                              sem_s, add=True) for k in range(KG)]
      for d in sds:
        d.wait()

    plsc.subcore_barrier()
    pltpu.sync_copy(acc.at[slab], agg_out.at[cid, slab])

  return sc_hop


_sc_hop = _make_sc_hop(combine=False)
_sc_hop_fused = _make_sc_hop(combine=True)


def _row_specs(n):
  return [pl.BlockSpec((BN, L), lambda i: (i, 0)) for _ in range(n)]


def _tc_pre(x_pad, wp, d0, d1):
  def body(x_ref, w_ref, d0_ref, d1_ref, u0_ref, inv_ref, dis_ref):
    deg = d0_ref[...] + d1_ref[...] + 1.0
    inv = 1.0 / deg
    dis = lax.rsqrt(deg)
    z = jnp.dot(x_ref[...], w_ref[...], preferred_element_type=jnp.float32)
    u0_ref[...] = dis * z
    inv_ref[...] = inv
    dis_ref[...] = dis

  return pl.pallas_call(
      body,
      grid=(N_PAD // BN,),
      in_specs=[pl.BlockSpec((BN, D), lambda i: (i, 0)),
                pl.BlockSpec((D, L), lambda i: (0, 0))] + _row_specs(2),
      out_specs=_row_specs(3),
      out_shape=[jax.ShapeDtypeStruct((N_PAD, L), jnp.float32)] * 3,
  )(x_pad, wp, d0, d1)


def _tc_mid(a0, a1, u0, inv):
  def body(a0_ref, a1_ref, u0_ref, inv_ref, u1_ref):
    u1_ref[...] = inv_ref[...] * (a0_ref[...] + a1_ref[...] + u0_ref[...])

  return pl.pallas_call(
      body,
      grid=(N_PAD // BN,),
      in_specs=_row_specs(4),
      out_specs=_row_specs(1)[0],
      out_shape=jax.ShapeDtypeStruct((N_PAD, L), jnp.float32),
  )(a0, a1, u0, inv)


def _tc_post(a0, a1, u1, dis, b16):
  def body(a0_ref, a1_ref, u1_ref, dis_ref, b_ref, o_ref):
    h2 = dis_ref[...] * (a0_ref[...] + a1_ref[...] + u1_ref[...])
    logits = h2 + b_ref[...]
    col = lax.broadcasted_iota(jnp.int32, (BN, L), 1)
    valid = col < C
    masked = jnp.where(valid, logits, jnp.float32(-1e30))
    m = jnp.max(masked, axis=1, keepdims=True)
    s = jnp.sum(jnp.where(valid, jnp.exp(logits - m), 0.0), axis=1,
                keepdims=True)
    o_ref[...] = logits - m - jnp.log(s)

  return pl.pallas_call(
      body,
      grid=(N_PAD // BN,),
      in_specs=_row_specs(3) + [pl.BlockSpec((BN, L), lambda i: (i, 0)),
                                pl.BlockSpec((1, L), lambda i: (0, 0))],
      out_specs=_row_specs(1)[0],
      out_shape=jax.ShapeDtypeStruct((N_PAD, L), jnp.float32),
  )(a0, a1, u1, dis, b16)


def kernel(x, edge_index, W, b):
  x_pad = jnp.pad(x, ((0, N_PAD - N), (0, 0)))
  wp = jnp.pad(W.T.astype(jnp.float32), ((0, 0), (0, L - C)))
  b16 = jnp.pad(b.astype(jnp.float32), (0, L - C)).reshape(1, L)
  src = jnp.pad(edge_index[0], (0, E_PAD - E), constant_values=N)
  dst = jnp.pad(edge_index[1], (0, E_PAD - E), constant_values=N)
  src3 = src.reshape(NW, CPT, CH)
  dst3 = dst.reshape(NW, CPT, CH)

  dummy_u = jnp.zeros((N_PAD, L), jnp.float32)
  degp = _sc_deg(dummy_u, src3, dst3)
  u0, inv, dis = _tc_pre(x_pad, wp, degp[0], degp[1])
  a1 = _sc_hop(u0, src3, dst3)
  a2, u1 = _sc_hop_fused(a1[0], a1[1], u0, inv, src3, dst3)
  out = _tc_post(a2[0], a2[1], u1, dis, b16)
  return out[:N, :C]


# speedup vs baseline: 53.6924x; 1.1967x over previous
import functools

import jax
import jax.numpy as jnp
from jax import lax
from jax.experimental import pallas as pl
from jax.experimental.pallas import tpu as pltpu
from jax.experimental.pallas import tpu_sc as plsc

N = 10000
E = 320000
D = 128
C = 7
L = 16
NC, NS = 2, 16
NW = NC * NS
EPT = E // NW
CH = 128
KG = 16
NCH = EPT // CH
NG_FULL = NCH // KG
REM = NCH - NG_FULL * KG
TAIL = EPT - NCH * CH
N_PAD = 10240
RPT = N_PAD // NS
BN = 512

_SC_PARAMS = pltpu.CompilerParams(use_tc_tiling_on_sc=False)
_MESH = plsc.VectorSubcoreMesh(core_axis_name="c", subcore_axis_name="s",
                               num_cores=NC, num_subcores=NS)


def _zero_acc_slab(rows_v, acc, sid):
  @pl.loop(0, CH)
  def _(i):
    rows_v[0, i, :] = jnp.zeros((L,), jnp.float32)

  @pl.loop(0, RPT // CH)
  def _(k):
    pltpu.sync_copy(rows_v.at[0], acc.at[pl.ds(sid * RPT + k * CH, CH)])


def _edge_sweep(gather_src, src_v, dst_v, rows_v, acc, sem_g, sem_s):
  def do_group(cbase, k_count):
    if gather_src is not None:
      gds = [pltpu.async_copy(
          gather_src.at[src_v.at[pl.ds((cbase + k) * CH, CH)]],
          rows_v.at[k], sem_g) for k in range(k_count)]
      for d in gds:
        d.wait()
    sds = [pltpu.async_copy(
        rows_v.at[k if gather_src is not None else 0],
        acc.at[dst_v.at[pl.ds((cbase + k) * CH, CH)]],
        sem_s, add=True) for k in range(k_count)]
    for d in sds:
      d.wait()

  @pl.loop(0, NG_FULL)
  def _(g):
    do_group(g * KG, KG)

  do_group(NG_FULL * KG, REM)

  if gather_src is not None:
    pltpu.async_copy(
        gather_src.at[src_v.at[pl.ds(NCH * CH, TAIL)]],
        rows_v.at[0, pl.ds(0, TAIL)], sem_g).wait()
  pltpu.async_copy(
      rows_v.at[0, pl.ds(0, TAIL)],
      acc.at[dst_v.at[pl.ds(NCH * CH, TAIL)]],
      sem_s, add=True).wait()


@functools.partial(
    pl.kernel,
    out_type=jax.ShapeDtypeStruct((NC, N_PAD, L), jnp.float32),
    mesh=_MESH,
    compiler_params=_SC_PARAMS,
    scratch_types=[
        pltpu.VMEM((EPT,), jnp.int32),
        pltpu.VMEM((KG, CH, L), jnp.float32),
        pltpu.VMEM_SHARED((N_PAD, L), jnp.float32),
        pltpu.SemaphoreType.DMA,
        pltpu.SemaphoreType.DMA,
    ],
)
def _sc_deg(ei_hbm, out_hbm, dst_v, rows_v, acc, sem_g, sem_s):
  cid = lax.axis_index("c")
  sid = lax.axis_index("s")
  wid = cid * NS + sid
  ds_dst = pltpu.async_copy(ei_hbm.at[1, pl.ds(wid * EPT, EPT)], dst_v, sem_s)

  _zero_acc_slab(rows_v, acc, sid)

  @pl.loop(0, CH)
  def _(i):
    rows_v[0, i, :] = jnp.ones((L,), jnp.float32)

  ds_dst.wait()
  plsc.subcore_barrier()
  _edge_sweep(None, None, dst_v, rows_v, acc, sem_g, sem_s)
  plsc.subcore_barrier()
  slab = pl.ds(sid * RPT, RPT)
  pltpu.sync_copy(acc.at[slab], out_hbm.at[cid, slab])


def _make_sc_hop(combine: bool):
  agg_t = jax.ShapeDtypeStruct((NC, N_PAD, L), jnp.float32)
  out_type = [agg_t, jax.ShapeDtypeStruct((N_PAD, L), jnp.float32)] \
      if combine else agg_t
  nwork = 4 if combine else 1

  @functools.partial(
      pl.kernel,
      out_type=out_type,
      mesh=_MESH,
      compiler_params=_SC_PARAMS,
      scratch_types=[
          pltpu.VMEM((EPT,), jnp.int32),
          pltpu.VMEM((EPT,), jnp.int32),
          pltpu.VMEM((KG, CH, L), jnp.float32),
          pltpu.VMEM((nwork, RPT, L), jnp.float32),
          pltpu.VMEM_SHARED((N_PAD, L), jnp.float32),
          pltpu.VMEM_SHARED((N_PAD, L), jnp.float32),
          pltpu.SemaphoreType.DMA,
          pltpu.SemaphoreType.DMA,
      ],
  )
  def sc_hop(*refs):
    if combine:
      (a0_hbm, a1_hbm, u_hbm, inv_hbm, ei_hbm, agg_out, u_out,
       src_v, dst_v, rows_v, work_v, u_spm, acc, sem_g, sem_s) = refs
    else:
      (u_hbm, ei_hbm, agg_out,
       src_v, dst_v, rows_v, work_v, u_spm, acc, sem_g, sem_s) = refs
    cid = lax.axis_index("c")
    sid = lax.axis_index("s")
    wid = cid * NS + sid
    slab = pl.ds(sid * RPT, RPT)

    ds_src = pltpu.async_copy(ei_hbm.at[0, pl.ds(wid * EPT, EPT)], src_v,
                              sem_g)
    ds_dst = pltpu.async_copy(ei_hbm.at[1, pl.ds(wid * EPT, EPT)], dst_v,
                              sem_s)

    if combine:
      pltpu.sync_copy(a0_hbm.at[slab], work_v.at[0])
      pltpu.sync_copy(a1_hbm.at[slab], work_v.at[1])
      pltpu.sync_copy(u_hbm.at[slab], work_v.at[2])
      pltpu.sync_copy(inv_hbm.at[slab], work_v.at[3])

      @pl.loop(0, RPT)
      def _(i):
        work_v[0, i, :] = work_v[3, i, :] * (
            work_v[0, i, :] + work_v[1, i, :] + work_v[2, i, :])

      pltpu.sync_copy(work_v.at[0], u_spm.at[slab])

      @pl.when(cid == 0)
      def _():
        pltpu.sync_copy(work_v.at[0], u_out.at[slab])
    else:
      pltpu.sync_copy(u_hbm.at[slab], u_spm.at[slab])

    _zero_acc_slab(rows_v, acc, sid)
    ds_src.wait()
    ds_dst.wait()
    plsc.subcore_barrier()
    _edge_sweep(u_spm, src_v, dst_v, rows_v, acc, sem_g, sem_s)
    plsc.subcore_barrier()
    pltpu.sync_copy(acc.at[slab], agg_out.at[cid, slab])

  return sc_hop


_sc_hop = _make_sc_hop(combine=False)
_sc_hop_fused = _make_sc_hop(combine=True)


def _row_specs(n):
  return [pl.BlockSpec((BN, L), lambda i: (i, 0)) for _ in range(n)]


def _tc_pre(x, wp, d0, d1):
  def body(x_ref, w_ref, d0_ref, d1_ref, u0_ref, inv_ref, dis_ref):
    deg = d0_ref[...] + d1_ref[...] + 1.0
    inv = 1.0 / deg
    dis = lax.rsqrt(deg)
    z = jnp.dot(x_ref[...], w_ref[...], preferred_element_type=jnp.float32)
    u0_ref[...] = dis * z
    inv_ref[...] = inv
    dis_ref[...] = dis

  return pl.pallas_call(
      body,
      grid=(N_PAD // BN,),
      in_specs=[pl.BlockSpec((BN, D), lambda i: (i, 0)),
                pl.BlockSpec((D, L), lambda i: (0, 0))] + _row_specs(2),
      out_specs=_row_specs(3),
      out_shape=[jax.ShapeDtypeStruct((N_PAD, L), jnp.float32)] * 3,
  )(x, wp, d0, d1)


def _tc_post(a0, a1, u1, dis, b16):
  def body(a0_ref, a1_ref, u1_ref, dis_ref, b_ref, o_ref):
    h2 = dis_ref[...] * (a0_ref[...] + a1_ref[...] + u1_ref[...])
    logits = h2 + b_ref[...]
    col = lax.broadcasted_iota(jnp.int32, (BN, L), 1)
    valid = col < C
    masked = jnp.where(valid, logits, jnp.float32(-1e30))
    m = jnp.max(masked, axis=1, keepdims=True)
    s = jnp.sum(jnp.where(valid, jnp.exp(logits - m), 0.0), axis=1,
                keepdims=True)
    o_ref[...] = logits - m - jnp.log(s)

  return pl.pallas_call(
      body,
      grid=(N_PAD // BN,),
      in_specs=_row_specs(3) + [pl.BlockSpec((BN, L), lambda i: (i, 0)),
                                pl.BlockSpec((1, L), lambda i: (0, 0))],
      out_specs=_row_specs(1)[0],
      out_shape=jax.ShapeDtypeStruct((N_PAD, L), jnp.float32),
  )(a0, a1, u1, dis, b16)


def kernel(x, edge_index, W, b):
  wp = jnp.pad(W.T.astype(jnp.float32), ((0, 0), (0, L - C)))
  b16 = jnp.pad(b.astype(jnp.float32), (0, L - C)).reshape(1, L)

  degp = _sc_deg(edge_index)
  u0, inv, dis = _tc_pre(x, wp, degp[0], degp[1])
  a1 = _sc_hop(u0, edge_index)
  a2, u1 = _sc_hop_fused(a1[0], a1[1], u0, inv, edge_index)
  out = _tc_post(a2[0], a2[1], u1, dis, b16)
  return out[:N, :C]
